# Initial kernel scaffold; baseline (speedup 1.0000x reference)
import jax
import jax.numpy as jnp
from jax.experimental import pallas as pl


def kernel(x, edge_index, edge_attr, batch, W1, b1, W2, b2, W3, b3, Wlin, blin):
    raise NotImplementedError("write your pallas kernel here")


# speedup vs baseline: 3.5135x; 3.5135x over previous
import functools

import jax
import jax.numpy as jnp
from jax import lax
from jax.experimental import pallas as pl
from jax.experimental.pallas import tpu as pltpu
from jax.experimental.pallas import tpu_sc as plsc

N = 10000
E = 320000
H = 128
G = 64
C_OUT = 2

NC, NS = 2, 16
NW = NC * NS
CHUNK = 128
EPW = 10240
E_PAD = EPW * NW
NCH = EPW // CHUNK
NROW = 10112
RPS = NROW // NS
BR = 1000


def _sc_mesh():
    return plsc.VectorSubcoreMesh(core_axis_name="c", subcore_axis_name="s")


@functools.cache
def _sc_pass(D):
    @functools.partial(
        pl.kernel,
        out_type=jax.ShapeDtypeStruct((NC * NROW, D), jnp.float32),
        mesh=_sc_mesh(),
        compiler_params=pltpu.CompilerParams(use_tc_tiling_on_sc=False),
        scratch_types=[
            pltpu.VMEM((CHUNK,), jnp.int32),
            pltpu.VMEM((CHUNK,), jnp.int32),
            pltpu.VMEM((CHUNK, D), jnp.float32),
            pltpu.VMEM((64, D), jnp.float32),
            pltpu.VMEM_SHARED((NROW, D), jnp.float32),
            pltpu.SemaphoreType.DMA,
        ],
    )
    def k(s_hbm, src_hbm, dst_hbm, z_hbm, out_hbm, src_v, dst_v, rows_v,
          st_v, acc, sem):
        c = lax.axis_index("c")
        sid = lax.axis_index("s")
        pltpu.sync_copy(z_hbm, st_v)
        r0 = sid * RPS
        nfull = RPS // 64
        rem = RPS - nfull * 64
        for j in range(nfull):
            pltpu.sync_copy(st_v, acc.at[pl.ds(r0 + j * 64, 64)])
        pltpu.sync_copy(st_v.at[pl.ds(0, rem)],
                        acc.at[pl.ds(r0 + nfull * 64, rem)])
        plsc.subcore_barrier()
        base = (sid * NC + c) * EPW

        def body(j, _):
            off = base + j * CHUNK
            pltpu.sync_copy(src_hbm.at[pl.ds(off, CHUNK)], src_v)
            pltpu.sync_copy(dst_hbm.at[pl.ds(off, CHUNK)], dst_v)
            pltpu.async_copy(s_hbm.at[src_v], rows_v, sem).wait()
            pltpu.sync_copy(rows_v, acc.at[dst_v], add=True)
            return 0

        lax.fori_loop(0, NCH, body, 0)
        plsc.subcore_barrier()
        out_base = c * NROW + r0
        for j in range(nfull):
            pltpu.sync_copy(acc.at[pl.ds(r0 + j * 64, 64)], st_v)
            pltpu.sync_copy(st_v, out_hbm.at[pl.ds(out_base + j * 64, 64)])
        pltpu.sync_copy(acc.at[pl.ds(r0 + nfull * 64, rem)],
                        st_v.at[pl.ds(0, rem)])
        pltpu.sync_copy(st_v.at[pl.ds(0, rem)],
                        out_hbm.at[pl.ds(out_base + nfull * 64, rem)])

    return k


@functools.cache
def _sc_deg():
    D = 16

    @functools.partial(
        pl.kernel,
        out_type=jax.ShapeDtypeStruct((NC * NROW, D), jnp.float32),
        mesh=_sc_mesh(),
        compiler_params=pltpu.CompilerParams(use_tc_tiling_on_sc=False),
        scratch_types=[
            pltpu.VMEM((CHUNK,), jnp.int32),
            pltpu.VMEM((CHUNK, D), jnp.float32),
            pltpu.VMEM((64, D), jnp.float32),
            pltpu.VMEM_SHARED((NROW, D), jnp.float32),
        ],
    )
    def k(dst_hbm, ones_hbm, z_hbm, out_hbm, dst_v, ones_v, st_v, acc):
        c = lax.axis_index("c")
        sid = lax.axis_index("s")
        pltpu.sync_copy(z_hbm, st_v)
        r0 = sid * RPS
        nfull = RPS // 64
        rem = RPS - nfull * 64
        for j in range(nfull):
            pltpu.sync_copy(st_v, acc.at[pl.ds(r0 + j * 64, 64)])
        pltpu.sync_copy(st_v.at[pl.ds(0, rem)],
                        acc.at[pl.ds(r0 + nfull * 64, rem)])
        pltpu.sync_copy(ones_hbm, ones_v)
        plsc.subcore_barrier()
        base = (sid * NC + c) * EPW

        def body(j, _):
            off = base + j * CHUNK
            pltpu.sync_copy(dst_hbm.at[pl.ds(off, CHUNK)], dst_v)
            pltpu.sync_copy(ones_v, acc.at[dst_v], add=True)
            return 0

        lax.fori_loop(0, NCH, body, 0)
        plsc.subcore_barrier()
        out_base = c * NROW + r0
        for j in range(nfull):
            pltpu.sync_copy(acc.at[pl.ds(r0 + j * 64, 64)], st_v)
            pltpu.sync_copy(st_v, out_hbm.at[pl.ds(out_base + j * 64, 64)])
        pltpu.sync_copy(acc.at[pl.ds(r0 + nfull * 64, rem)],
                        st_v.at[pl.ds(0, rem)])
        pltpu.sync_copy(st_v.at[pl.ds(0, rem)],
                        out_hbm.at[pl.ds(out_base + nfull * 64, rem)])

    return k


def _row_spec(d):
    return pl.BlockSpec((BR, d), lambda i: (i, 0))


def _full_spec(shape):
    return pl.BlockSpec(shape, lambda i: tuple(0 for _ in shape))


@functools.cache
def _prep():
    def body(da, db, xp, r_ref, s_ref):
        deg = da[...] + db[...]
        r = jnp.where(deg > 0.0, lax.rsqrt(jnp.maximum(deg, 1e-12)), 0.0)
        r_ref[...] = r
        s_ref[...] = xp[...] * r

    return pl.pallas_call(
        body,
        grid=(N // BR,),
        in_specs=[_row_spec(16), _row_spec(16), _row_spec(16)],
        out_specs=(_row_spec(16), _row_spec(16)),
        out_shape=(jax.ShapeDtypeStruct((N, 16), jnp.float32),
                   jax.ShapeDtypeStruct((N, 16), jnp.float32)),
    )


@functools.cache
def _combine(d_in, first, din0, bias_relu, emit, relu):
    def body(*refs):
        it = iter(refs)
        ta, tb, r16 = next(it), next(it), next(it)
        wk = next(it)
        if first:
            hh, w0 = next(it), next(it)
        else:
            acc = next(it)
        if bias_relu:
            b = next(it)
        out_ref = next(it)
        s_ref = next(it) if emit else None

        r = r16[...][:, 0:1]
        p = r * (ta[...] + tb[...])
        if first:
            base = jnp.dot(hh[...], w0[...], preferred_element_type=jnp.float32)
        else:
            base = acc[...]
        o = base + jnp.dot(p, wk[...], preferred_element_type=jnp.float32)
        if bias_relu:
            o = o + b[...]
            if relu:
                o = jnp.maximum(o, 0.0)
        out_ref[...] = o
        if emit == "p":
            s_ref[...] = r * p
        elif emit == "o":
            s_ref[...] = r * o

    in_specs = [_row_spec(d_in), _row_spec(d_in), _row_spec(16),
                _full_spec((d_in, H))]
    if first:
        in_specs += [_row_spec(din0), _full_spec((din0, H))]
    else:
        in_specs += [_row_spec(H)]
    if bias_relu:
        in_specs += [_full_spec((1, H))]
    out_shape = [jax.ShapeDtypeStruct((N, H), jnp.float32)]
    out_specs = [_row_spec(H)]
    if emit == "p":
        out_shape.append(jax.ShapeDtypeStruct((N, d_in), jnp.float32))
        out_specs.append(_row_spec(d_in))
    elif emit == "o":
        out_shape.append(jax.ShapeDtypeStruct((N, H), jnp.float32))
        out_specs.append(_row_spec(H))

    return pl.pallas_call(
        body,
        grid=(N // BR,),
        in_specs=in_specs,
        out_specs=tuple(out_specs),
        out_shape=tuple(out_shape),
    )


@functools.cache
def _pool():
    def body(h_ref, b_ref, sums_ref, cnt_ref):
        i = pl.program_id(0)
        oh = (b_ref[...] == lax.broadcasted_iota(jnp.int32, (1, G), 1))
        oh = oh.astype(jnp.float32)
        contrib = lax.dot_general(oh, h_ref[...],
                                  (((0,), (0,)), ((), ())),
                                  preferred_element_type=jnp.float32)
        cnt = lax.dot_general(oh, jnp.ones_like(h_ref[...]),
                              (((0,), (0,)), ((), ())),
                              preferred_element_type=jnp.float32)

        @pl.when(i == 0)
        def _():
            sums_ref[...] = jnp.zeros_like(sums_ref)
            cnt_ref[...] = jnp.zeros_like(cnt_ref)

        sums_ref[...] += contrib
        cnt_ref[...] += cnt

    return pl.pallas_call(
        body,
        grid=(N // BR,),
        in_specs=[_row_spec(H), _row_spec(1)],
        out_specs=(_full_spec((G, H)), _full_spec((G, H))),
        out_shape=(jax.ShapeDtypeStruct((G, H), jnp.float32),
                   jax.ShapeDtypeStruct((G, H), jnp.float32)),
    )


@functools.cache
def _final():
    def body(sums_ref, cnt_ref, w_ref, b_ref, out_ref):
        pooled = sums_ref[...] / jnp.maximum(cnt_ref[...], 1.0)
        logits = jnp.dot(pooled, w_ref[...],
                         preferred_element_type=jnp.float32) + b_ref[...]
        m = jnp.max(logits, axis=-1, keepdims=True)
        e = jnp.exp(logits - m)
        out_ref[...] = e / jnp.sum(e, axis=-1, keepdims=True)

    return pl.pallas_call(
        body,
        in_specs=[pl.BlockSpec((G, H), lambda: (0, 0)),
                  pl.BlockSpec((G, H), lambda: (0, 0)),
                  pl.BlockSpec((H, C_OUT), lambda: (0, 0)),
                  pl.BlockSpec((1, C_OUT), lambda: (0, 0))],
        out_specs=pl.BlockSpec((G, C_OUT), lambda: (0, 0)),
        out_shape=jax.ShapeDtypeStruct((G, C_OUT), jnp.float32),
    )


def kernel(x, edge_index, edge_attr, batch, W1, b1, W2, b2, W3, b3,
           Wlin, blin):
    del edge_attr
    src = edge_index[0]
    dst = edge_index[1]
    npad = E_PAD - E
    srcp = jnp.concatenate([src, jnp.zeros((npad,), jnp.int32)])
    dstp = jnp.concatenate([dst, jnp.full((npad,), N, jnp.int32)])
    z16 = jnp.zeros((64, 16), jnp.float32)
    z128 = jnp.zeros((64, H), jnp.float32)
    ones16 = jnp.ones((CHUNK, 16), jnp.float32)

    degp = _sc_deg()(dstp, ones16, z16)
    da = degp[:N]
    db = degp[NROW:NROW + N]

    xpad = jnp.pad(x, ((0, 0), (0, 12)))
    r16, s = _prep()(da, db, xpad)

    W1p = jnp.pad(W1, ((0, 0), (0, 12), (0, 0)))
    b1r = b1.reshape(1, H)
    b2r = b2.reshape(1, H)
    b3r = b3.reshape(1, H)

    pass16 = _sc_pass(16)
    pass128 = _sc_pass(H)

    t = pass16(s, srcp, dstp, z16)
    acc, s = _combine(16, True, 16, False, "p", False)(
        t[:N], t[NROW:NROW + N], r16, W1p[1], xpad, W1p[0])
    t = pass16(s, srcp, dstp, z16)
    h, s = _combine(16, False, 0, True, "o", True)(
        t[:N], t[NROW:NROW + N], r16, W1p[2], acc, b1r)

    t = pass128(s, srcp, dstp, z128)
    acc, s = _combine(H, True, H, False, "p", False)(
        t[:N], t[NROW:NROW + N], r16, W2[1], h, W2[0])
    t = pass128(s, srcp, dstp, z128)
    acc, s = _combine(H, False, 0, False, "p", False)(
        t[:N], t[NROW:NROW + N], r16, W2[2], acc)
    t = pass128(s, srcp, dstp, z128)
    h, s = _combine(H, False, 0, True, "o", True)(
        t[:N], t[NROW:NROW + N], r16, W2[3], acc, b2r)

    t = pass128(s, srcp, dstp, z128)
    acc, s = _combine(H, True, H, False, "p", False)(
        t[:N], t[NROW:NROW + N], r16, W3[1], h, W3[0])
    t = pass128(s, srcp, dstp, z128)
    acc, s = _combine(H, False, 0, False, "p", False)(
        t[:N], t[NROW:NROW + N], r16, W3[2], acc)
    t = pass128(s, srcp, dstp, z128)
    (h3,) = _combine(H, False, 0, True, None, False)(
        t[:N], t[NROW:NROW + N], r16, W3[3], acc, b3r)

    sums, cnt = _pool()(h3, batch.reshape(N, 1))
    return _final()(sums, cnt, Wlin, blin.reshape(1, C_OUT))


# speedup vs baseline: 4.5460x; 1.2939x over previous
import functools

import jax
import jax.numpy as jnp
from jax import lax
from jax.experimental import pallas as pl
from jax.experimental.pallas import tpu as pltpu
from jax.experimental.pallas import tpu_sc as plsc

N = 10000
E = 320000
H = 128
G = 64
C_OUT = 2

NC, NS = 2, 16
NW = NC * NS
CHUNK = 128
EPW = 10240
E_PAD = EPW * NW
NCH = EPW // CHUNK
NROW = 10112
RPS = NROW // NS
BR = 1000


def _sc_mesh():
    return plsc.VectorSubcoreMesh(core_axis_name="c", subcore_axis_name="s")


NBUF = 2
NIDX = 4
T_PIPE = NCH // NIDX
_ZCH = [128, 128, 128, 128, 120]
assert sum(_ZCH) == RPS


def _zero_acc(z_hbm, acc, r0):
    off = 0
    for n in _ZCH:
        pltpu.sync_copy(z_hbm.at[pl.ds(0, n)], acc.at[pl.ds(r0 + off, n)])
        off += n


def _copy_out(acc, out_hbm, r0, out_base):
    off = 0
    for n in _ZCH:
        pltpu.sync_copy(acc.at[pl.ds(r0 + off, n)],
                        out_hbm.at[pl.ds(out_base + off, n)])
        off += n


@functools.cache
def _sc_pass(D):
    @functools.partial(
        pl.kernel,
        out_type=jax.ShapeDtypeStruct((NC * NROW, D), jnp.float32),
        mesh=_sc_mesh(),
        compiler_params=pltpu.CompilerParams(use_tc_tiling_on_sc=False),
        scratch_types=[
            [pltpu.VMEM((2, CHUNK), jnp.int32) for _ in range(NIDX)],
            [pltpu.VMEM((CHUNK, D), jnp.float32) for _ in range(NBUF)],
            pltpu.VMEM_SHARED((NROW, D), jnp.float32),
            [pltpu.SemaphoreType.DMA for _ in range(NBUF)],
            [pltpu.SemaphoreType.DMA for _ in range(NIDX)],
        ],
    )
    def k(s_hbm, eidx_hbm, z_hbm, out_hbm, idx, rows, acc, g, si):
        c = lax.axis_index("c")
        sid = lax.axis_index("s")
        r0 = sid * RPS
        _zero_acc(z_hbm, acc, r0)
        w = sid * NC + c
        cb = w * NCH
        for b in range(NBUF):
            pltpu.sync_copy(eidx_hbm.at[cb + b], idx[b])
        for b in range(NBUF, NIDX):
            pltpu.async_copy(eidx_hbm.at[cb + b], idx[b], si[b])
        plsc.subcore_barrier()
        for b in range(NBUF):
            pltpu.async_copy(s_hbm.at[idx[b].at[0]], rows[b], g[b])

        def body(t, _):
            for b in range(NIDX):
                j = t * NIDX + b
                rb = b % NBUF
                bb = (b + NBUF) % NIDX
                pltpu.make_async_copy(s_hbm.at[pl.ds(0, CHUNK)], rows[rb],
                                      g[rb]).wait()
                pltpu.sync_copy(rows[rb], acc.at[idx[b].at[1]], add=True)
                pltpu.async_copy(eidx_hbm.at[cb + j + NIDX], idx[b], si[b])
                pltpu.make_async_copy(eidx_hbm.at[cb], idx[bb], si[bb]).wait()
                pltpu.async_copy(s_hbm.at[idx[bb].at[0]], rows[rb], g[rb])
            return 0

        lax.fori_loop(0, T_PIPE - 1, body, 0)
        for b in range(NIDX):
            j = (T_PIPE - 1) * NIDX + b
            rb = b % NBUF
            bb = (b + NBUF) % NIDX
            pltpu.make_async_copy(s_hbm.at[pl.ds(0, CHUNK)], rows[rb],
                                  g[rb]).wait()
            pltpu.sync_copy(rows[rb], acc.at[idx[b].at[1]], add=True)
            if j + NBUF < NCH:
                pltpu.make_async_copy(eidx_hbm.at[cb], idx[bb], si[bb]).wait()
                pltpu.async_copy(s_hbm.at[idx[bb].at[0]], rows[rb], g[rb])
        plsc.subcore_barrier()
        _copy_out(acc, out_hbm, r0, c * NROW + r0)

    return k


@functools.cache
def _sc_deg():
    D = 16

    @functools.partial(
        pl.kernel,
        out_type=jax.ShapeDtypeStruct((NC * NROW, D), jnp.float32),
        mesh=_sc_mesh(),
        compiler_params=pltpu.CompilerParams(use_tc_tiling_on_sc=False),
        scratch_types=[
            [pltpu.VMEM((2, CHUNK), jnp.int32) for _ in range(2)],
            pltpu.VMEM((CHUNK, D), jnp.float32),
            pltpu.VMEM_SHARED((NROW, D), jnp.float32),
            [pltpu.SemaphoreType.DMA for _ in range(2)],
        ],
    )
    def k(eidx_hbm, ones_hbm, z_hbm, out_hbm, idx, ones_v, acc, si):
        c = lax.axis_index("c")
        sid = lax.axis_index("s")
        r0 = sid * RPS
        _zero_acc(z_hbm, acc, r0)
        pltpu.sync_copy(ones_hbm, ones_v)
        w = sid * NC + c
        cb = w * NCH
        for b in range(2):
            pltpu.async_copy(eidx_hbm.at[cb + b], idx[b], si[b])
        plsc.subcore_barrier()

        def body(t, _):
            for b in range(2):
                j = 2 * t + b
                pltpu.make_async_copy(eidx_hbm.at[cb], idx[b], si[b]).wait()
                pltpu.sync_copy(ones_v, acc.at[idx[b].at[1]], add=True)
                pltpu.async_copy(eidx_hbm.at[cb + j + 2], idx[b], si[b])
            return 0

        lax.fori_loop(0, NCH // 2 - 1, body, 0)
        for b in range(2):
            pltpu.make_async_copy(eidx_hbm.at[cb], idx[b], si[b]).wait()
            pltpu.sync_copy(ones_v, acc.at[idx[b].at[1]], add=True)
        plsc.subcore_barrier()
        _copy_out(acc, out_hbm, r0, c * NROW + r0)

    return k


def _row_spec(d):
    return pl.BlockSpec((BR, d), lambda i: (i, 0))


def _full_spec(shape):
    return pl.BlockSpec(shape, lambda i: tuple(0 for _ in shape))


@functools.cache
def _prep():
    def body(da, db, xp, r_ref, s_ref):
        deg = da[...] + db[...]
        r = jnp.where(deg > 0.0, lax.rsqrt(jnp.maximum(deg, 1e-12)), 0.0)
        r_ref[...] = r
        s_ref[...] = xp[...] * r

    return pl.pallas_call(
        body,
        grid=(N // BR,),
        in_specs=[_row_spec(16), _row_spec(16), _row_spec(16)],
        out_specs=(_row_spec(16), _row_spec(16)),
        out_shape=(jax.ShapeDtypeStruct((N, 16), jnp.float32),
                   jax.ShapeDtypeStruct((N, 16), jnp.float32)),
    )


@functools.cache
def _combine(d_in, first, din0, bias_relu, emit, relu):
    def body(*refs):
        it = iter(refs)
        ta, tb, r16 = next(it), next(it), next(it)
        wk = next(it)
        if first:
            hh, w0 = next(it), next(it)
        else:
            acc = next(it)
        if bias_relu:
            b = next(it)
        out_ref = next(it)
        s_ref = next(it) if emit else None

        r = r16[...][:, 0:1]
        p = r * (ta[...] + tb[...])
        if first:
            base = jnp.dot(hh[...], w0[...], preferred_element_type=jnp.float32)
        else:
            base = acc[...]
        o = base + jnp.dot(p, wk[...], preferred_element_type=jnp.float32)
        if bias_relu:
            o = o + b[...]
            if relu:
                o = jnp.maximum(o, 0.0)
        out_ref[...] = o
        if emit == "p":
            s_ref[...] = r * p
        elif emit == "o":
            s_ref[...] = r * o

    in_specs = [_row_spec(d_in), _row_spec(d_in), _row_spec(16),
                _full_spec((d_in, H))]
    if first:
        in_specs += [_row_spec(din0), _full_spec((din0, H))]
    else:
        in_specs += [_row_spec(H)]
    if bias_relu:
        in_specs += [_full_spec((1, H))]
    out_shape = [jax.ShapeDtypeStruct((N, H), jnp.float32)]
    out_specs = [_row_spec(H)]
    if emit == "p":
        out_shape.append(jax.ShapeDtypeStruct((N, d_in), jnp.float32))
        out_specs.append(_row_spec(d_in))
    elif emit == "o":
        out_shape.append(jax.ShapeDtypeStruct((N, H), jnp.float32))
        out_specs.append(_row_spec(H))

    return pl.pallas_call(
        body,
        grid=(N // BR,),
        in_specs=in_specs,
        out_specs=tuple(out_specs),
        out_shape=tuple(out_shape),
    )


@functools.cache
def _pool():
    def body(h_ref, b_ref, sums_ref, cnt_ref):
        i = pl.program_id(0)
        oh = (b_ref[...] == lax.broadcasted_iota(jnp.int32, (1, G), 1))
        oh = oh.astype(jnp.float32)
        contrib = lax.dot_general(oh, h_ref[...],
                                  (((0,), (0,)), ((), ())),
                                  preferred_element_type=jnp.float32)
        cnt = lax.dot_general(oh, jnp.ones_like(h_ref[...]),
                              (((0,), (0,)), ((), ())),
                              preferred_element_type=jnp.float32)

        @pl.when(i == 0)
        def _():
            sums_ref[...] = jnp.zeros_like(sums_ref)
            cnt_ref[...] = jnp.zeros_like(cnt_ref)

        sums_ref[...] += contrib
        cnt_ref[...] += cnt

    return pl.pallas_call(
        body,
        grid=(N // BR,),
        in_specs=[_row_spec(H), _row_spec(1)],
        out_specs=(_full_spec((G, H)), _full_spec((G, H))),
        out_shape=(jax.ShapeDtypeStruct((G, H), jnp.float32),
                   jax.ShapeDtypeStruct((G, H), jnp.float32)),
    )


@functools.cache
def _final():
    def body(sums_ref, cnt_ref, w_ref, b_ref, out_ref):
        pooled = sums_ref[...] / jnp.maximum(cnt_ref[...], 1.0)
        logits = jnp.dot(pooled, w_ref[...],
                         preferred_element_type=jnp.float32) + b_ref[...]
        m = jnp.max(logits, axis=-1, keepdims=True)
        e = jnp.exp(logits - m)
        out_ref[...] = e / jnp.sum(e, axis=-1, keepdims=True)

    return pl.pallas_call(
        body,
        in_specs=[pl.BlockSpec((G, H), lambda: (0, 0)),
                  pl.BlockSpec((G, H), lambda: (0, 0)),
                  pl.BlockSpec((H, C_OUT), lambda: (0, 0)),
                  pl.BlockSpec((1, C_OUT), lambda: (0, 0))],
        out_specs=pl.BlockSpec((G, C_OUT), lambda: (0, 0)),
        out_shape=jax.ShapeDtypeStruct((G, C_OUT), jnp.float32),
    )


def kernel(x, edge_index, edge_attr, batch, W1, b1, W2, b2, W3, b3,
           Wlin, blin):
    del edge_attr
    src = edge_index[0]
    dst = edge_index[1]
    npad = E_PAD - E
    srcp = jnp.concatenate([src, jnp.zeros((npad,), jnp.int32)])
    dstp = jnp.concatenate([dst, jnp.full((npad,), N, jnp.int32)])
    eidx = jnp.stack([srcp.reshape(E_PAD // CHUNK, CHUNK),
                      dstp.reshape(E_PAD // CHUNK, CHUNK)], axis=1)
    z16 = jnp.zeros((CHUNK, 16), jnp.float32)
    z128 = jnp.zeros((CHUNK, H), jnp.float32)
    ones16 = jnp.ones((CHUNK, 16), jnp.float32)

    degp = _sc_deg()(eidx, ones16, z16)
    da = degp[:N]
    db = degp[NROW:NROW + N]

    xpad = jnp.pad(x, ((0, 0), (0, 12)))
    r16, s = _prep()(da, db, xpad)

    W1p = jnp.pad(W1, ((0, 0), (0, 12), (0, 0)))
    b1r = b1.reshape(1, H)
    b2r = b2.reshape(1, H)
    b3r = b3.reshape(1, H)

    pass16 = _sc_pass(16)
    pass128 = _sc_pass(H)

    t = pass16(s, eidx, z16)
    acc, s = _combine(16, True, 16, False, "p", False)(
        t[:N], t[NROW:NROW + N], r16, W1p[1], xpad, W1p[0])
    t = pass16(s, eidx, z16)
    h, s = _combine(16, False, 0, True, "o", True)(
        t[:N], t[NROW:NROW + N], r16, W1p[2], acc, b1r)

    t = pass128(s, eidx, z128)
    acc, s = _combine(H, True, H, False, "p", False)(
        t[:N], t[NROW:NROW + N], r16, W2[1], h, W2[0])
    t = pass128(s, eidx, z128)
    acc, s = _combine(H, False, 0, False, "p", False)(
        t[:N], t[NROW:NROW + N], r16, W2[2], acc)
    t = pass128(s, eidx, z128)
    h, s = _combine(H, False, 0, True, "o", True)(
        t[:N], t[NROW:NROW + N], r16, W2[3], acc, b2r)

    t = pass128(s, eidx, z128)
    acc, s = _combine(H, True, H, False, "p", False)(
        t[:N], t[NROW:NROW + N], r16, W3[1], h, W3[0])
    t = pass128(s, eidx, z128)
    acc, s = _combine(H, False, 0, False, "p", False)(
        t[:N], t[NROW:NROW + N], r16, W3[2], acc)
    t = pass128(s, eidx, z128)
    (h3,) = _combine(H, False, 0, True, None, False)(
        t[:N], t[NROW:NROW + N], r16, W3[3], acc, b3r)

    sums, cnt = _pool()(h3, batch.reshape(N, 1))
    return _final()(sums, cnt, Wlin, blin.reshape(1, C_OUT))
